# baseline (speedup 1.0000x reference)
import functools

import jax
import jax.numpy as jnp
from jax import lax
from jax.experimental import pallas as pl
from jax.experimental.pallas import tpu as pltpu
from jax.experimental.pallas import tpu_sc as plsc

N, E, F, H, DE = 10000, 320000, 128, 128, 16
TW = F + 16
NC, NS = 2, 16
NW = NC * NS
EPW = E // NW
CH = 80
NCHUNK = EPW // CH
NPS = N // NS
CHG = 200
NCHG = EPW // CHG

BN = 1000
BE = 2000


def _tables_body(h_ref, wcat_ref, cp_ref, ta_ref, tb_ref, s_ref):
    hw = jnp.dot(h_ref[...], wcat_ref[...], preferred_element_type=jnp.float32)
    cp = cp_ref[...]
    ta_ref[...] = jnp.concatenate(
        [hw[:, :F].astype(jnp.bfloat16), cp], axis=1)
    tb_ref[...] = jnp.concatenate(
        [hw[:, F:2 * F].astype(jnp.bfloat16), -cp], axis=1)
    s_ref[...] = hw[:, 2 * F:]


def _make_tables(h, wcat, cp):
    return pl.pallas_call(
        _tables_body,
        grid=(N // BN,),
        in_specs=[
            pl.BlockSpec((BN, F), lambda i: (i, 0)),
            pl.BlockSpec((F, 3 * F), lambda i: (0, 0)),
            pl.BlockSpec((BN, 16), lambda i: (i, 0)),
        ],
        out_specs=[
            pl.BlockSpec((BN, TW), lambda i: (i, 0)),
            pl.BlockSpec((BN, TW), lambda i: (i, 0)),
            pl.BlockSpec((BN, F), lambda i: (i, 0)),
        ],
        out_shape=[
            jax.ShapeDtypeStruct((N, TW), jnp.bfloat16),
            jax.ShapeDtypeStruct((N, TW), jnp.bfloat16),
            jax.ShapeDtypeStruct((N, F), jnp.float32),
        ],
    )(h, wcat, cp)


_sc_mesh = plsc.VectorSubcoreMesh(
    core_axis_name="c", subcore_axis_name="s", num_cores=NC, num_subcores=NS)


@functools.partial(
    pl.kernel,
    out_type=jax.ShapeDtypeStruct((E, TW), jnp.bfloat16),
    mesh=_sc_mesh,
    compiler_params=pltpu.CompilerParams(use_tc_tiling_on_sc=False),
    scratch_types=[
        pltpu.VMEM((CHG,), jnp.int32),
        pltpu.VMEM((CHG,), jnp.int32),
        pltpu.VMEM((CHG, TW), jnp.bfloat16),
    ],
)
def _sc_gather(ta_hbm, tb_hbm, row_hbm, col_hbm, g_hbm, ir, ic, av):
    wid = lax.axis_index("c") * NS + lax.axis_index("s")

    def chunk(j, carry):
        base = wid * EPW + j * CHG
        pltpu.sync_copy(row_hbm.at[pl.ds(base, CHG)], ir)
        pltpu.sync_copy(col_hbm.at[pl.ds(base, CHG)], ic)
        pltpu.sync_copy(ta_hbm.at[ir], av)
        pltpu.sync_copy(tb_hbm.at[ic], av, add=True)
        pltpu.sync_copy(av, g_hbm.at[pl.ds(base, CHG)])
        return carry

    lax.fori_loop(0, NCHG, chunk, 0)


def _edge_body(g_ref, ea_ref, em_ref, w1e_ref, w1r_ref, b1_ref,
               w2_ref, b2_ref, ef_ref):
    g = g_ref[...]
    gd = g[:, F:].astype(jnp.float32)
    radial = jnp.sum(gd * gd, axis=1, keepdims=True)
    pre = (g[:, :F].astype(jnp.float32) + radial * w1r_ref[...] + b1_ref[...]
           + jnp.dot(ea_ref[...], w1e_ref[...],
                     preferred_element_type=jnp.float32))
    m = pre * lax.logistic(pre)
    ef = jnp.dot(m, w2_ref[...], preferred_element_type=jnp.float32) + b2_ref[...]
    ef = ef * lax.logistic(ef)
    ef_ref[...] = (ef * em_ref[...]).astype(jnp.bfloat16)


def _edge_mlp(g, ea, em, w1e, w1r, b1r, w2, b2r):
    return pl.pallas_call(
        _edge_body,
        grid=(E // BE,),
        in_specs=[
            pl.BlockSpec((BE, TW), lambda i: (i, 0)),
            pl.BlockSpec((BE, DE), lambda i: (i, 0)),
            pl.BlockSpec((BE, 1), lambda i: (i, 0)),
            pl.BlockSpec((DE, H), lambda i: (0, 0)),
            pl.BlockSpec((1, H), lambda i: (0, 0)),
            pl.BlockSpec((1, H), lambda i: (0, 0)),
            pl.BlockSpec((H, H), lambda i: (0, 0)),
            pl.BlockSpec((1, H), lambda i: (0, 0)),
        ],
        out_specs=pl.BlockSpec((BE, H), lambda i: (i, 0)),
        out_shape=jax.ShapeDtypeStruct((E, H), jnp.bfloat16),
    )(g, ea, em, w1e, w1r, b1r, w2, b2r)


@functools.partial(
    pl.kernel,
    out_type=jax.ShapeDtypeStruct((NC, N, H), jnp.bfloat16),
    mesh=_sc_mesh,
    compiler_params=pltpu.CompilerParams(use_tc_tiling_on_sc=False),
    scratch_types=[
        pltpu.VMEM_SHARED((N, H), jnp.bfloat16),
        pltpu.VMEM((CH,), jnp.int32),
        pltpu.VMEM((CH, H), jnp.bfloat16),
    ],
)
def _sc_scatter(ef_hbm, row_hbm, zeros_hbm, agg_hbm, acc, iv, ev):
    cid = lax.axis_index("c")
    sid = lax.axis_index("s")
    wid = cid * NS + sid

    pltpu.sync_copy(zeros_hbm.at[pl.ds(sid * NPS, NPS)],
                    acc.at[pl.ds(sid * NPS, NPS)])
    plsc.subcore_barrier()

    def chunk(j, carry):
        base = wid * EPW + j * CH
        pltpu.sync_copy(row_hbm.at[pl.ds(base, CH)], iv)
        pltpu.sync_copy(ef_hbm.at[pl.ds(base, CH)], ev)
        pltpu.sync_copy(ev, acc.at[iv], add=True)
        return carry

    lax.fori_loop(0, NCHUNK, chunk, 0)
    plsc.subcore_barrier()
    pltpu.sync_copy(acc.at[pl.ds(sid * NPS, NPS)],
                    agg_hbm.at[cid, pl.ds(sid * NPS, NPS)])


def _node_body(s_ref, a0_ref, a1_ref, w3a_ref, b3_ref, w4_ref, b4_ref, o_ref):
    agg = a0_ref[...].astype(jnp.float32) + a1_ref[...].astype(jnp.float32)
    pre = (s_ref[...] + b3_ref[...]
           + jnp.dot(agg, w3a_ref[...], preferred_element_type=jnp.float32))
    t = pre * lax.logistic(pre)
    o_ref[...] = jnp.dot(t, w4_ref[...],
                         preferred_element_type=jnp.float32) + b4_ref[...]


def _node_mlp(s, a0, a1, w3a, b3r, w4, b4r):
    return pl.pallas_call(
        _node_body,
        grid=(N // BN,),
        in_specs=[
            pl.BlockSpec((BN, H), lambda i: (i, 0)),
            pl.BlockSpec((BN, H), lambda i: (i, 0)),
            pl.BlockSpec((BN, H), lambda i: (i, 0)),
            pl.BlockSpec((H, H), lambda i: (0, 0)),
            pl.BlockSpec((1, H), lambda i: (0, 0)),
            pl.BlockSpec((H, F), lambda i: (0, 0)),
            pl.BlockSpec((1, F), lambda i: (0, 0)),
        ],
        out_specs=pl.BlockSpec((BN, F), lambda i: (i, 0)),
        out_shape=jax.ShapeDtypeStruct((N, F), jnp.float32),
    )(s, a0, a1, w3a, b3r, w4, b4r)


def kernel(h, edge_index, coord, edge_mask, edge_attr,
           W1, b1, W2, b2, W3, b3, W4, b4):
    row = edge_index[0]
    col = edge_index[1]
    cp = jnp.pad(coord, ((0, 0), (0, 13))).astype(jnp.bfloat16)
    wcat = jnp.concatenate([W1[:F], W1[F:2 * F], W3[:F]], axis=1)

    ta, tb, s = _make_tables(h, wcat, cp)
    g = _sc_gather(ta, tb, row, col)
    ef = _edge_mlp(g, edge_attr, edge_mask,
                   W1[2 * F + 1:], W1[2 * F:2 * F + 1],
                   b1.reshape(1, H), W2, b2.reshape(1, H))
    agg = _sc_scatter(ef, row, jnp.zeros((N, H), jnp.bfloat16))
    h_out = _node_mlp(s, agg[0], agg[1], W3[F:], b3.reshape(1, H),
                      W4, b4.reshape(1, F))
    return (h_out, edge_attr)


# speedup vs baseline: 1.3067x; 1.3067x over previous
import functools

import jax
import jax.numpy as jnp
from jax import lax
from jax.experimental import pallas as pl
from jax.experimental.pallas import tpu as pltpu
from jax.experimental.pallas import tpu_sc as plsc

N, E, F, H, DE = 10000, 320000, 128, 128, 16
TW = F + 16
NC, NS = 2, 16
NW = NC * NS
EPW = E // NW
CH = 200
NPS = N // NS
HE = E // 2
HEPW = HE // NW
CHG = 200
NCHG = HEPW // CHG
NCHS = HEPW // CH

BN = 1000
BE = 2000


def _tables_body(h_ref, wcat_ref, cp_ref, ta_ref, tb_ref, s_ref):
    hw = jnp.dot(h_ref[...], wcat_ref[...], preferred_element_type=jnp.float32)
    cp = cp_ref[...]
    ta_ref[...] = jnp.concatenate(
        [hw[:, :F].astype(jnp.bfloat16), cp], axis=1)
    tb_ref[...] = jnp.concatenate(
        [hw[:, F:2 * F].astype(jnp.bfloat16), -cp], axis=1)
    s_ref[...] = hw[:, 2 * F:]


def _make_tables(h, wcat, cp):
    return pl.pallas_call(
        _tables_body,
        grid=(N // BN,),
        in_specs=[
            pl.BlockSpec((BN, F), lambda i: (i, 0)),
            pl.BlockSpec((F, 3 * F), lambda i: (0, 0)),
            pl.BlockSpec((BN, 16), lambda i: (i, 0)),
        ],
        out_specs=[
            pl.BlockSpec((BN, TW), lambda i: (i, 0)),
            pl.BlockSpec((BN, TW), lambda i: (i, 0)),
            pl.BlockSpec((BN, F), lambda i: (i, 0)),
        ],
        out_shape=[
            jax.ShapeDtypeStruct((N, TW), jnp.bfloat16),
            jax.ShapeDtypeStruct((N, TW), jnp.bfloat16),
            jax.ShapeDtypeStruct((N, F), jnp.float32),
        ],
    )(h, wcat, cp)


_sc_mesh = plsc.VectorSubcoreMesh(
    core_axis_name="c", subcore_axis_name="s", num_cores=NC, num_subcores=NS)


@functools.partial(
    pl.kernel,
    out_type=jax.ShapeDtypeStruct((HE, TW), jnp.bfloat16),
    mesh=_sc_mesh,
    compiler_params=pltpu.CompilerParams(use_tc_tiling_on_sc=False),
    scratch_types=[
        pltpu.VMEM((CHG,), jnp.int32),
        pltpu.VMEM((CHG,), jnp.int32),
        pltpu.VMEM((CHG, TW), jnp.bfloat16),
    ],
)
def _sc_gather(ta_hbm, tb_hbm, row_hbm, col_hbm, g_hbm, ir, ic, av):
    wid = lax.axis_index("c") * NS + lax.axis_index("s")

    def chunk(j, carry):
        base = wid * HEPW + j * CHG
        pltpu.sync_copy(row_hbm.at[pl.ds(base, CHG)], ir)
        pltpu.sync_copy(col_hbm.at[pl.ds(base, CHG)], ic)
        pltpu.sync_copy(ta_hbm.at[ir], av)
        pltpu.sync_copy(tb_hbm.at[ic], av, add=True)
        pltpu.sync_copy(av, g_hbm.at[pl.ds(base, CHG)])
        return carry

    lax.fori_loop(0, NCHG, chunk, 0)


def _edge_body(g_ref, ea_ref, em_ref, w1e_ref, w1r_ref, b1_ref,
               w2_ref, b2_ref, ef_ref):
    g = g_ref[...]
    gd = g[:, F:].astype(jnp.float32)
    radial = jnp.sum(gd * gd, axis=1, keepdims=True)
    pre = (g[:, :F].astype(jnp.float32) + radial * w1r_ref[...] + b1_ref[...]
           + jnp.dot(ea_ref[...], w1e_ref[...],
                     preferred_element_type=jnp.float32))
    m = pre * lax.logistic(pre)
    ef = jnp.dot(m, w2_ref[...], preferred_element_type=jnp.float32) + b2_ref[...]
    ef = ef * lax.logistic(ef)
    ef_ref[...] = ef * em_ref[...]


def _edge_mlp(g, ea, em, w1e, w1r, b1r, w2, b2r):
    return pl.pallas_call(
        _edge_body,
        grid=(HE // BE,),
        in_specs=[
            pl.BlockSpec((BE, TW), lambda i: (i, 0)),
            pl.BlockSpec((BE, DE), lambda i: (i, 0)),
            pl.BlockSpec((BE, 1), lambda i: (i, 0)),
            pl.BlockSpec((DE, H), lambda i: (0, 0)),
            pl.BlockSpec((1, H), lambda i: (0, 0)),
            pl.BlockSpec((1, H), lambda i: (0, 0)),
            pl.BlockSpec((H, H), lambda i: (0, 0)),
            pl.BlockSpec((1, H), lambda i: (0, 0)),
        ],
        out_specs=pl.BlockSpec((BE, H), lambda i: (i, 0)),
        out_shape=jax.ShapeDtypeStruct((HE, H), jnp.float32),
    )(g, ea, em, w1e, w1r, b1r, w2, b2r)


@functools.partial(
    pl.kernel,
    out_type=jax.ShapeDtypeStruct((NC, N, H), jnp.float32),
    mesh=_sc_mesh,
    compiler_params=pltpu.CompilerParams(use_tc_tiling_on_sc=False),
    scratch_types=[
        pltpu.VMEM_SHARED((N, H), jnp.float32),
        pltpu.VMEM((CH,), jnp.int32),
        pltpu.VMEM((CH, H), jnp.float32),
    ],
)
def _sc_scatter(ef_hbm, row_hbm, zeros_hbm, agg_hbm, acc, iv, ev):
    cid = lax.axis_index("c")
    sid = lax.axis_index("s")
    wid = cid * NS + sid

    pltpu.sync_copy(zeros_hbm.at[pl.ds(sid * NPS, NPS)],
                    acc.at[pl.ds(sid * NPS, NPS)])
    plsc.subcore_barrier()

    def chunk(j, carry):
        base = wid * HEPW + j * CH
        pltpu.sync_copy(row_hbm.at[pl.ds(base, CH)], iv)
        pltpu.sync_copy(ef_hbm.at[pl.ds(base, CH)], ev)
        pltpu.sync_copy(ev, acc.at[iv], add=True)
        return carry

    lax.fori_loop(0, NCHS, chunk, 0)
    plsc.subcore_barrier()
    pltpu.sync_copy(acc.at[pl.ds(sid * NPS, NPS)],
                    agg_hbm.at[cid, pl.ds(sid * NPS, NPS)])


def _node_body(s_ref, a0_ref, a1_ref, a2_ref, a3_ref,
               w3a_ref, b3_ref, w4_ref, b4_ref, o_ref):
    agg = (a0_ref[...] + a1_ref[...]) + (a2_ref[...] + a3_ref[...])
    pre = (s_ref[...] + b3_ref[...]
           + jnp.dot(agg, w3a_ref[...], preferred_element_type=jnp.float32))
    t = pre * lax.logistic(pre)
    o_ref[...] = jnp.dot(t, w4_ref[...],
                         preferred_element_type=jnp.float32) + b4_ref[...]


def _node_mlp(s, a0, a1, a2, a3, w3a, b3r, w4, b4r):
    return pl.pallas_call(
        _node_body,
        grid=(N // BN,),
        in_specs=[
            pl.BlockSpec((BN, H), lambda i: (i, 0)),
            pl.BlockSpec((BN, H), lambda i: (i, 0)),
            pl.BlockSpec((BN, H), lambda i: (i, 0)),
            pl.BlockSpec((BN, H), lambda i: (i, 0)),
            pl.BlockSpec((BN, H), lambda i: (i, 0)),
            pl.BlockSpec((H, H), lambda i: (0, 0)),
            pl.BlockSpec((1, H), lambda i: (0, 0)),
            pl.BlockSpec((H, F), lambda i: (0, 0)),
            pl.BlockSpec((1, F), lambda i: (0, 0)),
        ],
        out_specs=pl.BlockSpec((BN, F), lambda i: (i, 0)),
        out_shape=jax.ShapeDtypeStruct((N, F), jnp.float32),
    )(s, a0, a1, a2, a3, w3a, b3r, w4, b4r)


def kernel(h, edge_index, coord, edge_mask, edge_attr,
           W1, b1, W2, b2, W3, b3, W4, b4):
    row = edge_index[0]
    col = edge_index[1]
    cp = jnp.pad(coord, ((0, 0), (0, 13))).astype(jnp.bfloat16)
    wcat = jnp.concatenate([W1[:F], W1[F:2 * F], W3[:F]], axis=1)

    ta, tb, s = _make_tables(h, wcat, cp)
    w1e = W1[2 * F + 1:]
    w1r = W1[2 * F:2 * F + 1]
    b1r = b1.reshape(1, H)
    b2r = b2.reshape(1, H)
    zer = jnp.zeros((N, H), jnp.float32)

    g0 = _sc_gather(ta, tb, row[:HE], col[:HE])
    g1 = _sc_gather(ta, tb, row[HE:], col[HE:])
    ef0 = _edge_mlp(g0, edge_attr[:HE], edge_mask[:HE], w1e, w1r, b1r, W2, b2r)
    ef1 = _edge_mlp(g1, edge_attr[HE:], edge_mask[HE:], w1e, w1r, b1r, W2, b2r)
    agg0 = _sc_scatter(ef0, row[:HE], zer)
    agg1 = _sc_scatter(ef1, row[HE:], zer)
    h_out = _node_mlp(s, agg0[0], agg0[1], agg1[0], agg1[1],
                      W3[F:], b3.reshape(1, H), W4, b4.reshape(1, F))
    return (h_out, edge_attr)
